# baseline (speedup 1.0000x reference)
import functools

import jax
import jax.numpy as jnp
from jax.experimental import pallas as pl
from jax.experimental.pallas import tpu as pltpu
from jax.experimental.pallas import tpu_sc as plsc

NUM_NODE = 1024
TIME_DIM = 32

NC = 2
NS = 16
LANES = 16

NW = NC * NS
FILL_LANES = 1024
NBUF = 2


def _sc_broadcast_lookup(time_flat, table_padded, n_pairs):
    pairs_per_w = n_pairs // NW
    dmas_per_pair = NUM_NODE // FILL_LANES
    chunks = FILL_LANES // LANES

    mesh = plsc.VectorSubcoreMesh(
        core_axis_name="c", subcore_axis_name="s",
        num_cores=NC, num_subcores=NS,
    )

    @functools.partial(
        pl.kernel,
        out_type=jax.ShapeDtypeStruct((n_pairs * TIME_DIM, NUM_NODE),
                                      jnp.float32),
        mesh=mesh,
        scratch_types=[
            pltpu.VMEM((pairs_per_w,), jnp.int32),
            pltpu.VMEM((pairs_per_w, 128), jnp.float32),
            pltpu.VMEM((NBUF * TIME_DIM, FILL_LANES), jnp.float32),
            pltpu.SemaphoreType.DMA,
            pltpu.SemaphoreType.DMA,
            pltpu.SemaphoreType.DMA,
        ],
    )
    def k(time_hbm, table_hbm, out_hbm, idx_v, rows_v, rep, gsem, ws0, ws1):
        wid = jax.lax.axis_index("s") * NC + jax.lax.axis_index("c")
        base = wid * pairs_per_w

        pltpu.sync_copy(time_hbm.at[pl.ds(base, pairs_per_w)], idx_v)
        pltpu.async_copy(table_hbm.at[idx_v], rows_v, gsem).wait()

        wsems = [ws0, ws1]
        inflight = [None] * NBUF
        for i in range(pairs_per_w):
            b = i % NBUF
            if inflight[b] is not None:
                for d in inflight[b]:
                    d.wait()

            v_lo = rows_v[i, pl.ds(0, LANES)]
            v_hi = rows_v[i, pl.ds(LANES, LANES)]
            splats = []
            for d in range(TIME_DIM):
                v, ld = (v_lo, d) if d < LANES else (v_hi, d - LANES)
                s = jax.lax.squeeze(jax.lax.slice(v, (ld,), (ld + 1,)), (0,))
                splats.append(jax.lax.broadcast_in_dim(s, (LANES,), ()))
            sect = b * TIME_DIM

            @pl.loop(0, chunks)
            def _fill(c):
                col = c * LANES
                for d in range(TIME_DIM):
                    rep[sect + d, pl.ds(col, LANES)] = splats[d]

            src = rep.at[pl.ds(sect, TIME_DIM)]
            out_row = (base + i) * TIME_DIM
            inflight[b] = [
                pltpu.async_copy(
                    src,
                    out_hbm.at[pl.ds(out_row, TIME_DIM),
                               pl.ds(j * FILL_LANES, FILL_LANES)],
                    wsems[b])
                for j in range(dmas_per_pair)
            ]
        for dmas in inflight:
            if dmas is not None:
                for d in dmas:
                    d.wait()

    return k


def kernel(time, weekday, time_table):
    del weekday
    batch, seq, _ = time.shape
    n_pairs = batch * seq
    time_flat = time.reshape(n_pairs).astype(jnp.int32)
    table_padded = jnp.pad(time_table, ((0, 0), (0, 128 - TIME_DIM)))
    out = _sc_broadcast_lookup(time_flat, table_padded, n_pairs)(
        time_flat, table_padded)
    out = out.reshape(batch, seq, TIME_DIM, NUM_NODE)
    return out.transpose(0, 1, 3, 2)


# speedup vs baseline: 1.0304x; 1.0304x over previous
import functools

import jax
import jax.numpy as jnp
from jax.experimental import pallas as pl
from jax.experimental.pallas import tpu as pltpu
from jax.experimental.pallas import tpu_sc as plsc

NUM_NODE = 1024
TIME_DIM = 32

NC = 2
NS = 16
LANES = 16

NW = NC * NS
FILL_LANES = 1024
NBUF = 2


def _sc_broadcast_lookup(time_flat, table_padded, n_pairs):
    pairs_per_w = n_pairs // NW
    dmas_per_pair = NUM_NODE // FILL_LANES
    chunks = FILL_LANES // LANES

    mesh = plsc.VectorSubcoreMesh(
        core_axis_name="c", subcore_axis_name="s",
        num_cores=NC, num_subcores=NS,
    )

    @functools.partial(
        pl.kernel,
        out_type=jax.ShapeDtypeStruct((n_pairs * TIME_DIM, NUM_NODE),
                                      jnp.float32),
        mesh=mesh,
        compiler_params=pltpu.CompilerParams(skip_device_barrier=True),
        scratch_types=[
            pltpu.VMEM((pairs_per_w,), jnp.int32),
            pltpu.VMEM((pairs_per_w, 128), jnp.float32),
            pltpu.VMEM((NBUF * TIME_DIM, FILL_LANES), jnp.float32),
            pltpu.SemaphoreType.DMA,
            pltpu.SemaphoreType.DMA,
            pltpu.SemaphoreType.DMA,
        ],
    )
    def k(time_hbm, table_hbm, out_hbm, idx_v, rows_v, rep, gsem, ws0, ws1):
        wid = jax.lax.axis_index("s") * NC + jax.lax.axis_index("c")
        base = wid * pairs_per_w

        pltpu.sync_copy(time_hbm.at[pl.ds(base, pairs_per_w)], idx_v)
        pltpu.async_copy(table_hbm.at[idx_v], rows_v, gsem).wait()

        wsems = [ws0, ws1]
        inflight = [None] * NBUF
        for i in range(pairs_per_w):
            b = i % NBUF
            if inflight[b] is not None:
                for d in inflight[b]:
                    d.wait()

            v_lo = rows_v[i, pl.ds(0, LANES)]
            v_hi = rows_v[i, pl.ds(LANES, LANES)]
            splats = []
            for d in range(TIME_DIM):
                v, ld = (v_lo, d) if d < LANES else (v_hi, d - LANES)
                s = jax.lax.squeeze(jax.lax.slice(v, (ld,), (ld + 1,)), (0,))
                splats.append(jax.lax.broadcast_in_dim(s, (LANES,), ()))
            sect = b * TIME_DIM

            @pl.loop(0, chunks)
            def _fill(c):
                col = c * LANES
                for d in range(TIME_DIM):
                    rep[sect + d, pl.ds(col, LANES)] = splats[d]

            src = rep.at[pl.ds(sect, TIME_DIM)]
            out_row = (base + i) * TIME_DIM
            inflight[b] = [
                pltpu.async_copy(
                    src,
                    out_hbm.at[pl.ds(out_row, TIME_DIM),
                               pl.ds(j * FILL_LANES, FILL_LANES)],
                    wsems[b])
                for j in range(dmas_per_pair)
            ]
        for dmas in inflight:
            if dmas is not None:
                for d in dmas:
                    d.wait()

    return k


def kernel(time, weekday, time_table):
    del weekday
    batch, seq, _ = time.shape
    n_pairs = batch * seq
    time_flat = time.reshape(n_pairs).astype(jnp.int32)
    table_padded = jnp.pad(time_table, ((0, 0), (0, 128 - TIME_DIM)))
    out = _sc_broadcast_lookup(time_flat, table_padded, n_pairs)(
        time_flat, table_padded)
    out = out.reshape(batch, seq, TIME_DIM, NUM_NODE)
    return out.transpose(0, 1, 3, 2)
